# baseline (speedup 1.0000x reference)
import jax
import jax.numpy as jnp
from jax import lax
from jax.experimental import pallas as pl
from jax.experimental.pallas import tpu as pltpu

_TB = 2048


def _fused_pool_fc_kernel(x_ref, w_ref, out_ref):
    x = x_ref[...]
    w = w_ref[...]
    w2 = jnp.stack([w, jnp.zeros_like(w)], axis=-1).reshape(w.shape[0], -1)
    pooled_full = jnp.maximum(x, pltpu.roll(x, x.shape[1] - 1, 1))
    out_ref[...] = lax.dot_general(
        pooled_full, w2,
        dimension_numbers=(((1,), (1,)), ((), ())),
        preferred_element_type=jnp.float32)


def kernel(x, fc1_weight):
    Bx, C, L = x.shape
    n_classes, K = fc1_weight.shape
    xflat = x.reshape(Bx, C * L)

    tb = min(_TB, Bx)
    grid = (pl.cdiv(Bx, tb),)
    cost = pl.CostEstimate(
        flops=2 * Bx * K * n_classes + Bx * C * L,
        transcendentals=0,
        bytes_accessed=4 * (Bx * C * L + n_classes * K + Bx * n_classes),
    )
    return pl.pallas_call(
        _fused_pool_fc_kernel,
        out_shape=jax.ShapeDtypeStruct((Bx, n_classes), jnp.float32),
        grid=grid,
        in_specs=[pl.BlockSpec((tb, C * L), lambda b: (b, 0)),
                  pl.BlockSpec((n_classes, K), lambda b: (0, 0))],
        out_specs=pl.BlockSpec((tb, n_classes), lambda b: (b, 0)),
        compiler_params=pltpu.CompilerParams(dimension_semantics=("parallel",)),
        cost_estimate=cost,
    )(xflat, fc1_weight)


# speedup vs baseline: 1.0076x; 1.0076x over previous
import jax
import jax.numpy as jnp
from jax import lax
from jax.experimental import pallas as pl
from jax.experimental.pallas import tpu as pltpu

_TB = 4096


def _fused_pool_fc_kernel(x_ref, w_ref, out_ref):
    x = x_ref[...]
    w = w_ref[...]
    w2 = jnp.stack([w, jnp.zeros_like(w)], axis=-1).reshape(w.shape[0], -1)
    pooled_full = jnp.maximum(x, pltpu.roll(x, x.shape[1] - 1, 1))
    out_ref[...] = lax.dot_general(
        pooled_full, w2,
        dimension_numbers=(((1,), (1,)), ((), ())),
        preferred_element_type=jnp.float32)


def kernel(x, fc1_weight):
    Bx, C, L = x.shape
    n_classes, K = fc1_weight.shape
    xflat = x.reshape(Bx, C * L)

    tb = min(_TB, Bx)
    grid = (pl.cdiv(Bx, tb),)
    cost = pl.CostEstimate(
        flops=2 * Bx * K * n_classes + Bx * C * L,
        transcendentals=0,
        bytes_accessed=4 * (Bx * C * L + n_classes * K + Bx * n_classes),
    )
    return pl.pallas_call(
        _fused_pool_fc_kernel,
        out_shape=jax.ShapeDtypeStruct((Bx, n_classes), jnp.float32),
        grid=grid,
        in_specs=[pl.BlockSpec((tb, C * L), lambda b: (b, 0)),
                  pl.BlockSpec((n_classes, K), lambda b: (0, 0))],
        out_specs=pl.BlockSpec((tb, n_classes), lambda b: (b, 0)),
        compiler_params=pltpu.CompilerParams(dimension_semantics=("parallel",)),
        cost_estimate=cost,
    )(xflat, fc1_weight)
